# Initial kernel scaffold; baseline (speedup 1.0000x reference)
import jax
import jax.numpy as jnp
from jax.experimental import pallas as pl


def kernel(feat, edge_index, W1, W2):
    raise NotImplementedError("write your pallas kernel here")


# speedup vs baseline: 5.6322x; 5.6322x over previous
import jax
import jax.numpy as jnp
from jax import lax
from jax.experimental import pallas as pl
from jax.experimental.pallas import tpu as pltpu
from jax.experimental.pallas import tpu_sc as plsc

N_NODES = 10000
N_EDGES = 320000
D = 128
DH = D // 2

NC = 2
NS = 16

EDGES_PER_TILE = N_EDGES // NS
CHUNK = 80
CHUNKS_PER_TILE = EDGES_PER_TILE // CHUNK

N_PAD = 10240
ROWS_PER_TILE = N_PAD // NS
COPY_ROWS = 128
COPY_STEPS = ROWS_PER_TILE // COPY_ROWS


def _sc_agg_body(feat_hbm, src_hbm, dst_hbm, out_hbm,
                 src_v, dst_v, rows_v, stage_v, agg_sh, gsem):
    c = lax.axis_index("c")
    s = lax.axis_index("s")

    zeros16 = jnp.zeros((16,), jnp.float32)

    def _zero_row(i, carry):
        for j in range(DH // 16):
            stage_v[i, pl.ds(j * 16, 16)] = zeros16
        return carry

    lax.fori_loop(0, COPY_ROWS, _zero_row, 0)
    for p in range(COPY_STEPS):
        pltpu.sync_copy(stage_v, agg_sh.at[pl.ds(s * ROWS_PER_TILE + p * COPY_ROWS, COPY_ROWS)])

    pltpu.sync_copy(src_hbm.at[s], src_v)
    pltpu.sync_copy(dst_hbm.at[s], dst_v)

    plsc.subcore_barrier()

    def _edge_chunk(i, carry):
        pltpu.async_copy(feat_hbm.at[c].at[src_v.at[i]], rows_v, gsem).wait()
        pltpu.sync_copy(rows_v, agg_sh.at[dst_v.at[i]], add=True)
        return carry

    lax.fori_loop(0, CHUNKS_PER_TILE, _edge_chunk, 0)

    plsc.subcore_barrier()

    for p in range(COPY_STEPS):
        base = s * ROWS_PER_TILE + p * COPY_ROWS
        pltpu.sync_copy(agg_sh.at[pl.ds(base, COPY_ROWS)], stage_v)
        pltpu.sync_copy(stage_v, out_hbm.at[c, pl.ds(base, COPY_ROWS)])


@jax.jit
def _sc_aggregate(feat_sp, src_r, dst_r):
    mesh = plsc.VectorSubcoreMesh(core_axis_name="c", subcore_axis_name="s")
    return pl.kernel(
        _sc_agg_body,
        out_type=jax.ShapeDtypeStruct((NC, N_PAD, DH), jnp.float32),
        mesh=mesh,
        scratch_types=[
            pltpu.VMEM((CHUNKS_PER_TILE, CHUNK), jnp.int32),
            pltpu.VMEM((CHUNKS_PER_TILE, CHUNK), jnp.int32),
            pltpu.VMEM((CHUNK, DH), jnp.float32),
            pltpu.VMEM((COPY_ROWS, DH), jnp.float32),
            pltpu.VMEM_SHARED((N_PAD, DH), jnp.float32),
            pltpu.SemaphoreType.DMA,
        ],
        compiler_params=pltpu.CompilerParams(use_tc_tiling_on_sc=False),
    )(feat_sp, src_r, dst_r)


def _tc_mm_body(feat_ref, p_ref, w1_ref, w2_ref, out_ref):
    agg = jnp.concatenate([p_ref[0], p_ref[1]], axis=-1)
    out_ref[...] = (
        jnp.dot(feat_ref[...], w1_ref[...], preferred_element_type=jnp.float32)
        + jnp.dot(agg, w2_ref[...], preferred_element_type=jnp.float32)
    )


@jax.jit
def _tc_matmul(feat, partials, W1, W2):
    blk = 1000
    grid = (N_NODES // blk,)
    return pl.pallas_call(
        _tc_mm_body,
        grid=grid,
        in_specs=[
            pl.BlockSpec((blk, D), lambda i: (i, 0)),
            pl.BlockSpec((NC, blk, DH), lambda i: (0, i, 0)),
            pl.BlockSpec((D, D), lambda i: (0, 0)),
            pl.BlockSpec((D, D), lambda i: (0, 0)),
        ],
        out_specs=pl.BlockSpec((blk, D), lambda i: (i, 0)),
        out_shape=jax.ShapeDtypeStruct((N_NODES, D), jnp.float32),
    )(feat, partials, W1, W2)


def kernel(feat, edge_index, W1, W2):
    edge_index = edge_index.astype(jnp.int32)
    src_r = edge_index[0].reshape(NS, CHUNKS_PER_TILE, CHUNK)
    dst_r = edge_index[1].reshape(NS, CHUNKS_PER_TILE, CHUNK)
    feat_sp = jnp.stack([feat[:, :DH], feat[:, DH:]])
    partials = _sc_aggregate(feat_sp, src_r, dst_r)
    return _tc_matmul(feat, partials, W1, W2)


# speedup vs baseline: 8.0416x; 1.4278x over previous
import jax
import jax.numpy as jnp
from jax import lax
from jax.experimental import pallas as pl
from jax.experimental.pallas import tpu as pltpu
from jax.experimental.pallas import tpu_sc as plsc

N_NODES = 10000
N_EDGES = 320000
D = 128
DH = D // 2

NC = 2
NS = 16

EDGES_PER_TILE = N_EDGES // NS
CHUNK = 125
CHUNKS_PER_TILE = EDGES_PER_TILE // CHUNK

N_PAD = 10240
ROWS_PER_TILE = N_PAD // NS
COPY_ROWS = 128
COPY_STEPS = ROWS_PER_TILE // COPY_ROWS


def _sc_agg_body(feat_hbm, src_hbm, dst_hbm, out_hbm,
                 src_v, dst_v, rows0, rows1, stage_v, agg_sh, gsem0, gsem1):
    c = lax.axis_index("c")
    s = lax.axis_index("s")

    zeros16 = jnp.zeros((16,), jnp.float32)

    def _zero_row(i, carry):
        for j in range(DH // 16):
            stage_v[i, pl.ds(j * 16, 16)] = zeros16
        return carry

    lax.fori_loop(0, COPY_ROWS, _zero_row, 0)
    for p in range(COPY_STEPS):
        pltpu.sync_copy(stage_v, agg_sh.at[pl.ds(s * ROWS_PER_TILE + p * COPY_ROWS, COPY_ROWS)])

    pltpu.sync_copy(src_hbm.at[s], src_v)
    pltpu.sync_copy(dst_hbm.at[s], dst_v)

    plsc.subcore_barrier()

    table = feat_hbm.at[c]

    def _gather(i, buf, sem):
        pltpu.async_copy(table.at[src_v.at[i]], buf, sem)

    def _gwait(buf, sem):
        pltpu.make_async_copy(table.at[src_v.at[0]], buf, sem).wait()

    last = CHUNKS_PER_TILE - 1
    _gather(0, rows0, gsem0)

    def _pair(i, carry):
        i0 = 2 * i
        _gwait(rows0, gsem0)
        _gather(i0 + 1, rows1, gsem1)
        pltpu.sync_copy(rows0, agg_sh.at[dst_v.at[i0]], add=True)
        _gwait(rows1, gsem1)
        _gather(jnp.minimum(i0 + 2, last), rows0, gsem0)
        pltpu.sync_copy(rows1, agg_sh.at[dst_v.at[i0 + 1]], add=True)
        return carry

    lax.fori_loop(0, CHUNKS_PER_TILE // 2, _pair, 0)
    _gwait(rows0, gsem0)

    plsc.subcore_barrier()

    for p in range(COPY_STEPS):
        base = s * ROWS_PER_TILE + p * COPY_ROWS
        pltpu.sync_copy(agg_sh.at[pl.ds(base, COPY_ROWS)], stage_v)
        pltpu.sync_copy(stage_v, out_hbm.at[c, pl.ds(base, COPY_ROWS)])


@jax.jit
def _sc_aggregate(feat_sp, src_r, dst_r):
    mesh = plsc.VectorSubcoreMesh(core_axis_name="c", subcore_axis_name="s")
    return pl.kernel(
        _sc_agg_body,
        out_type=jax.ShapeDtypeStruct((NC, N_PAD, DH), jnp.float32),
        mesh=mesh,
        scratch_types=[
            pltpu.VMEM((CHUNKS_PER_TILE, CHUNK), jnp.int32),
            pltpu.VMEM((CHUNKS_PER_TILE, CHUNK), jnp.int32),
            pltpu.VMEM((CHUNK, DH), jnp.float32),
            pltpu.VMEM((CHUNK, DH), jnp.float32),
            pltpu.VMEM((COPY_ROWS, DH), jnp.float32),
            pltpu.VMEM_SHARED((N_PAD, DH), jnp.float32),
            pltpu.SemaphoreType.DMA,
            pltpu.SemaphoreType.DMA,
        ],
        compiler_params=pltpu.CompilerParams(use_tc_tiling_on_sc=False),
    )(feat_sp, src_r, dst_r)


def _tc_mm_body(feat_ref, p_ref, w1_ref, w2_ref, out_ref):
    agg = jnp.concatenate([p_ref[0], p_ref[1]], axis=-1)
    out_ref[...] = (
        jnp.dot(feat_ref[...], w1_ref[...], preferred_element_type=jnp.float32)
        + jnp.dot(agg, w2_ref[...], preferred_element_type=jnp.float32)
    )


@jax.jit
def _tc_matmul(feat, partials, W1, W2):
    blk = 1000
    grid = (N_NODES // blk,)
    return pl.pallas_call(
        _tc_mm_body,
        grid=grid,
        in_specs=[
            pl.BlockSpec((blk, D), lambda i: (i, 0)),
            pl.BlockSpec((NC, blk, DH), lambda i: (0, i, 0)),
            pl.BlockSpec((D, D), lambda i: (0, 0)),
            pl.BlockSpec((D, D), lambda i: (0, 0)),
        ],
        out_specs=pl.BlockSpec((blk, D), lambda i: (i, 0)),
        out_shape=jax.ShapeDtypeStruct((N_NODES, D), jnp.float32),
    )(feat, partials, W1, W2)


def kernel(feat, edge_index, W1, W2):
    edge_index = edge_index.astype(jnp.int32)
    src_r = edge_index[0].reshape(NS, CHUNKS_PER_TILE, CHUNK)
    dst_r = edge_index[1].reshape(NS, CHUNKS_PER_TILE, CHUNK)
    feat_sp = jnp.stack([feat[:, :DH], feat[:, DH:]])
    partials = _sc_aggregate(feat_sp, src_r, dst_r)
    return _tc_matmul(feat, partials, W1, W2)


# speedup vs baseline: 11.5586x; 1.4374x over previous
import jax
import jax.numpy as jnp
from jax import lax
from jax.experimental import pallas as pl
from jax.experimental.pallas import tpu as pltpu
from jax.experimental.pallas import tpu_sc as plsc

N_NODES = 10000
N_EDGES = 320000
D = 128
DH = D // 2

NC = 2
NS = 16

EDGES_PER_TILE = N_EDGES // NS
CHUNK = 125
CHUNKS_PER_TILE = EDGES_PER_TILE // CHUNK
NBUF = 4

N_PAD = 10240
ROWS_PER_TILE = N_PAD // NS
COPY_ROWS = 128
COPY_STEPS = ROWS_PER_TILE // COPY_ROWS


def _sc_agg_body(feat_hbm, src_hbm, dst_hbm, out_hbm,
                 src_v, dst_v, rows, stage_v, agg_sh, gsems, ssems):
    c = lax.axis_index("c")
    s = lax.axis_index("s")

    zeros16 = jnp.zeros((16,), jnp.float32)

    def _zero_row(i, carry):
        for j in range(DH // 16):
            stage_v[i, pl.ds(j * 16, 16)] = zeros16
        return carry

    lax.fori_loop(0, COPY_ROWS, _zero_row, 0)
    for p in range(COPY_STEPS):
        pltpu.sync_copy(stage_v, agg_sh.at[pl.ds(s * ROWS_PER_TILE + p * COPY_ROWS, COPY_ROWS)])

    pltpu.sync_copy(src_hbm.at[c, s], src_v)
    pltpu.sync_copy(dst_hbm.at[s], dst_v)

    plsc.subcore_barrier()

    def _gstart(i, b):
        pltpu.async_copy(feat_hbm.at[src_v.at[i]], rows[b], gsems[b])

    def _gwait(b):
        pltpu.make_async_copy(feat_hbm.at[src_v.at[0]], rows[b], gsems[b]).wait()

    def _sstart(i, b):
        pltpu.async_copy(rows[b], agg_sh.at[dst_v.at[i]], ssems[b], add=True)

    def _swait(b):
        pltpu.make_async_copy(rows[b], agg_sh.at[dst_v.at[0]], ssems[b]).wait()

    last = CHUNKS_PER_TILE - 1
    for b in range(NBUF):
        _gstart(b, b)

    def _group(g, carry):
        base = NBUF * g
        for b in range(NBUF):
            _gwait(b)
            _sstart(base + b, b)
        for b in range(NBUF):
            _swait(b)
            _gstart(jnp.minimum(base + NBUF + b, last), b)
        return carry

    lax.fori_loop(0, CHUNKS_PER_TILE // NBUF, _group, 0)
    for b in range(NBUF):
        _gwait(b)

    plsc.subcore_barrier()

    for p in range(COPY_STEPS):
        base = s * ROWS_PER_TILE + p * COPY_ROWS
        pltpu.sync_copy(agg_sh.at[pl.ds(base, COPY_ROWS)], stage_v)
        pltpu.sync_copy(stage_v, out_hbm.at[c, pl.ds(base, COPY_ROWS)])


@jax.jit
def _sc_aggregate(feat2, src2_r, dst_r):
    mesh = plsc.VectorSubcoreMesh(core_axis_name="c", subcore_axis_name="s")
    return pl.kernel(
        _sc_agg_body,
        out_type=jax.ShapeDtypeStruct((NC, N_PAD, DH), jnp.float32),
        mesh=mesh,
        scratch_types=[
            pltpu.VMEM((CHUNKS_PER_TILE, CHUNK), jnp.int32),
            pltpu.VMEM((CHUNKS_PER_TILE, CHUNK), jnp.int32),
            [pltpu.VMEM((CHUNK, DH), jnp.float32)] * NBUF,
            pltpu.VMEM((COPY_ROWS, DH), jnp.float32),
            pltpu.VMEM_SHARED((N_PAD, DH), jnp.float32),
            [pltpu.SemaphoreType.DMA] * NBUF,
            [pltpu.SemaphoreType.DMA] * NBUF,
        ],
        compiler_params=pltpu.CompilerParams(use_tc_tiling_on_sc=False),
    )(feat2, src2_r, dst_r)


def _tc_mm_body(feat_ref, p_ref, w1_ref, w2_ref, out_ref):
    agg = jnp.concatenate([p_ref[0], p_ref[1]], axis=-1)
    out_ref[...] = (
        jnp.dot(feat_ref[...], w1_ref[...], preferred_element_type=jnp.float32)
        + jnp.dot(agg, w2_ref[...], preferred_element_type=jnp.float32)
    )


@jax.jit
def _tc_matmul(feat, partials, W1, W2):
    blk = 1000
    grid = (N_NODES // blk,)
    return pl.pallas_call(
        _tc_mm_body,
        grid=grid,
        in_specs=[
            pl.BlockSpec((blk, D), lambda i: (i, 0)),
            pl.BlockSpec((NC, blk, DH), lambda i: (0, i, 0)),
            pl.BlockSpec((D, D), lambda i: (0, 0)),
            pl.BlockSpec((D, D), lambda i: (0, 0)),
        ],
        out_specs=pl.BlockSpec((blk, D), lambda i: (i, 0)),
        out_shape=jax.ShapeDtypeStruct((N_NODES, D), jnp.float32),
    )(feat, partials, W1, W2)


def kernel(feat, edge_index, W1, W2):
    edge_index = edge_index.astype(jnp.int32)
    src2 = edge_index[0] * 2
    src2_r = (src2[None, :] + jnp.arange(NC, dtype=jnp.int32)[:, None]).reshape(
        NC, NS, CHUNKS_PER_TILE, CHUNK)
    dst_r = edge_index[1].reshape(NS, CHUNKS_PER_TILE, CHUNK)
    feat2 = feat.reshape(N_NODES * 2, DH)
    partials = _sc_aggregate(feat2, src2_r, dst_r)
    return _tc_matmul(feat, partials, W1, W2)
